# Initial kernel scaffold; baseline (speedup 1.0000x reference)
import functools
import math

import jax
import jax.numpy as jnp
from jax import lax
from jax.experimental import pallas as pl
from jax.experimental.pallas import tpu as pltpu
from jax.experimental.pallas import tpu_sc as plsc

_CHUNK = 128


def _scale_body(x_ref, o_ref, *, scale):
    o_ref[...] = x_ref[...] * scale


def _scale_table(table, scale):
    v, d = table.shape
    blk = 2500
    assert v % blk == 0
    return pl.pallas_call(
        functools.partial(_scale_body, scale=scale),
        grid=(v // blk,),
        in_specs=[pl.BlockSpec((blk, d), lambda i: (i, 0))],
        out_specs=pl.BlockSpec((blk, d), lambda i: (i, 0)),
        out_shape=jax.ShapeDtypeStruct((v, d), table.dtype),
    )(table)


@functools.cache
def _make_gather(nc, ns, n_chunks, chunk, d):
    nw = nc * ns
    b_per_w = n_chunks * chunk
    n_pairs = n_chunks // 2
    assert n_chunks % 2 == 0
    mesh = plsc.VectorSubcoreMesh(core_axis_name="c", subcore_axis_name="s")

    @functools.partial(
        pl.kernel,
        mesh=mesh,
        out_type=jax.ShapeDtypeStruct((nw * b_per_w, d), jnp.float32),
        scratch_types=[
            pltpu.VMEM((n_chunks, chunk), jnp.int32),
            pltpu.VMEM((chunk, d), jnp.float32),
            pltpu.VMEM((chunk, d), jnp.float32),
            pltpu.SemaphoreType.DMA,
            pltpu.SemaphoreType.DMA,
        ],
    )
    def gather(table_hbm, idx_hbm, out_hbm, idx_v, buf0, buf1, gsem0, gsem1):
        wid = lax.axis_index("s") * nc + lax.axis_index("c")
        base = wid * b_per_w
        pltpu.sync_copy(idx_hbm.at[wid], idx_v)
        pltpu.async_copy(table_hbm.at[idx_v.at[0]], buf0, gsem0)

        def pair(i, carry):
            j0 = 2 * i
            pltpu.async_copy(table_hbm.at[idx_v.at[j0 + 1]], buf1, gsem1)
            pltpu.make_async_copy(table_hbm.at[idx_v.at[j0]], buf0, gsem0).wait()
            pltpu.sync_copy(buf0, out_hbm.at[pl.ds(base + j0 * chunk, chunk)])

            @pl.when(i + 1 < n_pairs)
            def _():
                pltpu.async_copy(table_hbm.at[idx_v.at[j0 + 2]], buf0, gsem0)

            pltpu.make_async_copy(table_hbm.at[idx_v.at[j0 + 1]], buf1, gsem1).wait()
            pltpu.sync_copy(buf1, out_hbm.at[pl.ds(base + (j0 + 1) * chunk, chunk)])
            return carry

        lax.fori_loop(0, n_pairs, pair, 0)

    return gather


def kernel(tokens, table):
    v, d = table.shape
    b, s = tokens.shape
    n = b * s
    info = plsc.get_sparse_core_info()
    nc, ns = info.num_cores, info.num_subcores
    nw = nc * ns
    assert n % (nw * _CHUNK) == 0
    n_chunks = n // (nw * _CHUNK)

    scaled = _scale_table(table, math.sqrt(d))
    idx = tokens.reshape(nw, n_chunks, _CHUNK).astype(jnp.int32)
    out = _make_gather(nc, ns, n_chunks, _CHUNK, d)(scaled, idx)
    return out.reshape(b, s, d)


# speedup vs baseline: 7.9862x; 7.9862x over previous
import functools
import math

import jax
import jax.numpy as jnp
from jax import lax
from jax.experimental import pallas as pl
from jax.experimental.pallas import tpu as pltpu
from jax.experimental.pallas import tpu_sc as plsc

_CHUNK = 128


def _scale_body(x_ref, o_ref, *, scale):
    o_ref[...] = x_ref[...] * scale


def _scale_table(table, scale):
    v, d = table.shape
    blk = 2000
    assert v % blk == 0
    return pl.pallas_call(
        functools.partial(_scale_body, scale=scale),
        grid=(v // blk,),
        in_specs=[pl.BlockSpec((blk, d), lambda i: (i, 0))],
        out_specs=pl.BlockSpec((blk, d), lambda i: (i, 0)),
        out_shape=jax.ShapeDtypeStruct((v, d), table.dtype),
    )(table)


@functools.cache
def _make_gather(nc, ns, n_chunks, chunk, d):
    nw = nc * ns
    b_per_w = n_chunks * chunk
    n_pairs = n_chunks // 2
    assert n_chunks % 2 == 0
    mesh = plsc.VectorSubcoreMesh(core_axis_name="c", subcore_axis_name="s")

    @functools.partial(
        pl.kernel,
        mesh=mesh,
        out_type=jax.ShapeDtypeStruct((nw * b_per_w, d), jnp.float32),
        scratch_types=[
            pltpu.VMEM((n_chunks, chunk), jnp.int32),
            pltpu.VMEM((chunk, d), jnp.float32),
            pltpu.VMEM((chunk, d), jnp.float32),
            pltpu.SemaphoreType.DMA,
            pltpu.SemaphoreType.DMA,
        ],
    )
    def gather(table_hbm, idx_hbm, out_hbm, idx_v, buf0, buf1, gsem0, gsem1):
        wid = lax.axis_index("s") * nc + lax.axis_index("c")
        base = wid * b_per_w
        pltpu.sync_copy(idx_hbm.at[wid], idx_v)
        pltpu.async_copy(table_hbm.at[idx_v.at[0]], buf0, gsem0)

        def pair(i, carry):
            j0 = 2 * i
            pltpu.async_copy(table_hbm.at[idx_v.at[j0 + 1]], buf1, gsem1)
            pltpu.make_async_copy(table_hbm.at[idx_v.at[j0]], buf0, gsem0).wait()
            pltpu.sync_copy(buf0, out_hbm.at[pl.ds(base + j0 * chunk, chunk)])

            @pl.when(i + 1 < n_pairs)
            def _():
                pltpu.async_copy(table_hbm.at[idx_v.at[j0 + 2]], buf0, gsem0)

            pltpu.make_async_copy(table_hbm.at[idx_v.at[j0 + 1]], buf1, gsem1).wait()
            pltpu.sync_copy(buf1, out_hbm.at[pl.ds(base + (j0 + 1) * chunk, chunk)])
            return carry

        lax.fori_loop(0, n_pairs, pair, 0)

    return gather


def kernel(tokens, table):
    v, d = table.shape
    b, s = tokens.shape
    n = b * s
    info = plsc.get_sparse_core_info()
    nc, ns = info.num_cores, info.num_subcores
    nw = nc * ns
    assert n % (nw * _CHUNK) == 0
    n_chunks = n // (nw * _CHUNK)

    scaled = _scale_table(table, math.sqrt(d))
    idx = tokens.reshape(nw, n_chunks, _CHUNK).astype(jnp.int32)
    out = _make_gather(nc, ns, n_chunks, _CHUNK, d)(scaled, idx)
    return out.reshape(b, s, d)


# speedup vs baseline: 9.0787x; 1.1368x over previous
import functools
import math

import jax
import jax.numpy as jnp
from jax import lax
from jax.experimental import pallas as pl
from jax.experimental.pallas import tpu as pltpu
from jax.experimental.pallas import tpu_sc as plsc

_CHUNK = 128
_GPB = 2
_NBUF = 2
_LANES = 16


@functools.cache
def _make_gather(nc, ns, n_chunks, chunk, d, scale):
    nw = nc * ns
    b_per_w = n_chunks * chunk
    brows = _GPB * chunk
    n_bufchunks = n_chunks // _GPB
    assert n_chunks % (_GPB * _NBUF) == 0
    n_groups = n_bufchunks // _NBUF
    n_vec = d // _LANES
    mesh = plsc.VectorSubcoreMesh(core_axis_name="c", subcore_axis_name="s")

    @functools.partial(
        pl.kernel,
        mesh=mesh,
        out_type=jax.ShapeDtypeStruct((nw * b_per_w, d), jnp.float32),
        scratch_types=[
            pltpu.VMEM((n_chunks, chunk), jnp.int32),
        ]
        + [pltpu.VMEM((brows, d), jnp.float32) for _ in range(_NBUF)]
        + [pltpu.SemaphoreType.DMA for _ in range(2 * _NBUF)],
    )
    def gather(table_hbm, idx_hbm, out_hbm, idx_v, *rest):
        bufs = rest[:_NBUF]
        gsems = rest[_NBUF : 2 * _NBUF]
        wsems = rest[2 * _NBUF :]
        wid = lax.axis_index("s") * nc + lax.axis_index("c")
        base = wid * b_per_w

        def fire_gathers(k, b):
            for p in range(_GPB):
                pltpu.async_copy(
                    table_hbm.at[idx_v.at[_GPB * k + p]],
                    bufs[b].at[pl.ds(p * chunk, chunk)],
                    gsems[b],
                )

        def wait_gathers(k, b):
            for p in range(_GPB):
                pltpu.make_async_copy(
                    table_hbm.at[idx_v.at[_GPB * k + p]],
                    bufs[b].at[pl.ds(p * chunk, chunk)],
                    gsems[b],
                ).wait()

        def out_slice(k):
            return out_hbm.at[pl.ds(base + k * brows, brows)]

        pltpu.sync_copy(idx_hbm.at[wid], idx_v)
        for b in range(_NBUF):
            fire_gathers(b, b)

        def scale_buf(buf):
            def row(r, carry):
                for rr in range(2):
                    for c in range(n_vec):
                        sl = pl.ds(c * _LANES, _LANES)
                        buf[2 * r + rr, sl] = buf[2 * r + rr, sl] * scale
                return carry

            lax.fori_loop(0, brows // 2, row, 0)

        def group(g, carry):
            k0 = g * _NBUF
            for b in range(_NBUF):
                wait_gathers(k0 + b, b)
                scale_buf(bufs[b])
                pltpu.async_copy(bufs[b], out_slice(k0 + b), wsems[b])

            @pl.when(g < n_groups - 1)
            def _():
                for b in range(_NBUF):
                    pltpu.make_async_copy(bufs[b], out_slice(k0 + b), wsems[b]).wait()
                    fire_gathers(k0 + _NBUF + b, b)

            return carry

        lax.fori_loop(0, n_groups, group, 0)

        last = (n_groups - 1) * _NBUF
        for b in range(_NBUF):
            pltpu.make_async_copy(bufs[b], out_slice(last + b), wsems[b]).wait()

    return gather


def kernel(tokens, table):
    v, d = table.shape
    b, s = tokens.shape
    n = b * s
    info = plsc.get_sparse_core_info()
    nc, ns = info.num_cores, info.num_subcores
    nw = nc * ns
    assert n % (nw * _CHUNK) == 0 and d % _LANES == 0
    n_chunks = n // (nw * _CHUNK)

    idx = tokens.reshape(nw, n_chunks, _CHUNK).astype(jnp.int32)
    out = _make_gather(nc, ns, n_chunks, _CHUNK, d, math.sqrt(d))(table, idx)
    return out.reshape(b, s, d)


# speedup vs baseline: 9.1246x; 1.0051x over previous
import functools
import math

import jax
import jax.numpy as jnp
from jax import lax
from jax.experimental import pallas as pl
from jax.experimental.pallas import tpu as pltpu
from jax.experimental.pallas import tpu_sc as plsc

_CHUNK = 128
_GPB = 1
_NBUF = 5
_LANES = 16


@functools.cache
def _make_gather(nc, ns, n_chunks, chunk, d, scale):
    nw = nc * ns
    b_per_w = n_chunks * chunk
    brows = _GPB * chunk
    n_bufchunks = n_chunks // _GPB
    assert n_chunks % (_GPB * _NBUF) == 0
    n_groups = n_bufchunks // _NBUF
    n_vec = d // _LANES
    mesh = plsc.VectorSubcoreMesh(core_axis_name="c", subcore_axis_name="s")

    @functools.partial(
        pl.kernel,
        mesh=mesh,
        out_type=jax.ShapeDtypeStruct((nw * b_per_w, d), jnp.float32),
        scratch_types=[
            pltpu.VMEM((n_chunks, chunk), jnp.int32),
        ]
        + [pltpu.VMEM((brows, d), jnp.float32) for _ in range(_NBUF)]
        + [pltpu.SemaphoreType.DMA for _ in range(2 * _NBUF)],
    )
    def gather(table_hbm, idx_hbm, out_hbm, idx_v, *rest):
        bufs = rest[:_NBUF]
        gsems = rest[_NBUF : 2 * _NBUF]
        wsems = rest[2 * _NBUF :]
        wid = lax.axis_index("s") * nc + lax.axis_index("c")
        base = wid * b_per_w

        def fire_gathers(k, b):
            for p in range(_GPB):
                pltpu.async_copy(
                    table_hbm.at[idx_v.at[_GPB * k + p]],
                    bufs[b].at[pl.ds(p * chunk, chunk)],
                    gsems[b],
                )

        def wait_gathers(k, b):
            for p in range(_GPB):
                pltpu.make_async_copy(
                    table_hbm.at[idx_v.at[_GPB * k + p]],
                    bufs[b].at[pl.ds(p * chunk, chunk)],
                    gsems[b],
                ).wait()

        def out_slice(k):
            return out_hbm.at[pl.ds(base + k * brows, brows)]

        pltpu.sync_copy(idx_hbm.at[wid], idx_v)
        for b in range(_NBUF):
            fire_gathers(b, b)

        def scale_buf(buf):
            def row(r, carry):
                for rr in range(2):
                    for c in range(n_vec):
                        sl = pl.ds(c * _LANES, _LANES)
                        buf[2 * r + rr, sl] = buf[2 * r + rr, sl] * scale
                return carry

            lax.fori_loop(0, brows // 2, row, 0)

        def group(g, carry):
            k0 = g * _NBUF
            for b in range(_NBUF):
                wait_gathers(k0 + b, b)
                scale_buf(bufs[b])
                pltpu.async_copy(bufs[b], out_slice(k0 + b), wsems[b])

            @pl.when(g < n_groups - 1)
            def _():
                for b in range(_NBUF):
                    pltpu.make_async_copy(bufs[b], out_slice(k0 + b), wsems[b]).wait()
                    fire_gathers(k0 + _NBUF + b, b)

            return carry

        lax.fori_loop(0, n_groups, group, 0)

        last = (n_groups - 1) * _NBUF
        for b in range(_NBUF):
            pltpu.make_async_copy(bufs[b], out_slice(last + b), wsems[b]).wait()

    return gather


def kernel(tokens, table):
    v, d = table.shape
    b, s = tokens.shape
    n = b * s
    info = plsc.get_sparse_core_info()
    nc, ns = info.num_cores, info.num_subcores
    nw = nc * ns
    assert n % (nw * _CHUNK) == 0 and d % _LANES == 0
    n_chunks = n // (nw * _CHUNK)

    idx = tokens.reshape(nw, n_chunks, _CHUNK).astype(jnp.int32)
    out = _make_gather(nc, ns, n_chunks, _CHUNK, d, math.sqrt(d))(table, idx)
    return out.reshape(b, s, d)
